# baseline (speedup 1.0000x reference)
import jax
import jax.numpy as jnp
from jax import lax
from jax.experimental import pallas as pl
from jax.experimental.pallas import tpu as pltpu
from jax.experimental.pallas import tpu_sc as plsc

_V = 100000
_B = 1024
_VA = 99968
_VT = _V - _VA
_C = 1408
_NCH = _VA // _C
_NV = _C // 16
_G = 16
_NG = 2
_RPT = _G * _NG

_SC_PARAMS = pltpu.CompilerParams(needs_layout_passes=False)


def _mesh():
    return plsc.VectorSubcoreMesh(core_axis_name="c", subcore_axis_name="s")


def _s_body(idx_hbm, prow_hbm, w1_hbm, table_hbm, s_hbm,
            idx_v, prow_v, w1_v, table_v, out_v):
    @pl.when((lax.axis_index("c") == 0) & (lax.axis_index("s") == 0))
    def _():
        pltpu.sync_copy(idx_hbm, idx_v)
        pltpu.sync_copy(prow_hbm, prow_v)
        pltpu.sync_copy(w1_hbm, w1_v)
        pltpu.sync_copy(table_hbm, table_v)
        acc = jnp.zeros((16,), jnp.float32)
        for h in range(2):
            iv = idx_v[pl.ds(h * 16, 16)]
            vals = plsc.load_gather(table_v, [iv])
            acc = acc + vals * prow_v[pl.ds(h * 16, 16)]
        tot = jnp.sum(acc)
        out_v[...] = tot * w1_v[...]
        pltpu.sync_copy(out_v, s_hbm)


def _s_sc(idx, prow, w1b, table):
    fn = pl.kernel(
        _s_body,
        out_type=jax.ShapeDtypeStruct((16,), jnp.float32),
        mesh=_mesh(),
        scratch_types=[
            pltpu.VMEM((32,), jnp.int32),
            pltpu.VMEM((32,), jnp.float32),
            pltpu.VMEM((16,), jnp.float32),
            pltpu.VMEM((_V,), jnp.float32),
            pltpu.VMEM((16,), jnp.float32),
        ],
        compiler_params=_SC_PARAMS,
    )
    return fn(idx, prow, w1b, table)


def _p1_chunk(xb, yb, fb, sv, mstat, lstat, b, grp):
    for r in range(_G):
        lr = grp * _G + r
        macc = mstat[pl.ds(lr * 16, 16)]

        def sw1(v, mv):
            xv = xb[b, r, pl.ds(v * 16, 16)]
            fv = fb[b, pl.ds(v * 16, 16)]
            yv = xv + sv * fv
            yb[b, r, pl.ds(v * 16, 16)] = yv
            return jnp.maximum(mv, yv)

        mnew = lax.fori_loop(0, _NV, sw1, macc)

        def sw2(v, acc):
            yv = yb[b, r, pl.ds(v * 16, 16)]
            return acc + jnp.exp(yv - mnew)

        lsum = lax.fori_loop(0, _NV, sw2, jnp.zeros((16,), jnp.float32))
        lold = lstat[pl.ds(lr * 16, 16)]
        lstat[pl.ds(lr * 16, 16)] = lold * jnp.exp(macc - mnew) + lsum
        mstat[pl.ds(lr * 16, 16)] = mnew


def _p1_body(x_hbm, xt_hbm, f_hbm, ft_hbm, s_hbm, m_hbm, l_hbm,
             xb, yb, fb, sb, xtb, ftb, mstat, lstat, xsem, fsem):
    wid = lax.axis_index("s") * 2 + lax.axis_index("c")
    row0 = wid * _RPT
    pltpu.sync_copy(s_hbm, sb)
    sv = sb[...]
    for lr in range(_RPT):
        mstat[pl.ds(lr * 16, 16)] = jnp.full((16,), -3.0e38, jnp.float32)
        lstat[pl.ds(lr * 16, 16)] = jnp.zeros((16,), jnp.float32)

    for grp in range(_NG):
        rowg = row0 + grp * _G

        def x_copy(c, b):
            return pltpu.make_async_copy(
                x_hbm.at[pl.ds(rowg, _G), pl.ds(c * _C, _C)],
                xb.at[b], xsem.at[b])

        def f_copy(c, b):
            return pltpu.make_async_copy(
                f_hbm.at[pl.ds(c * _C, _C)], fb.at[b], fsem.at[b])

        x_copy(0, 0).start()
        f_copy(0, 0).start()
        x_copy(1, 1).start()
        f_copy(1, 1).start()

        def pair(t, carry):
            for b in range(2):
                c = 2 * t + b
                x_copy(c, b).wait()
                f_copy(c, b).wait()
                _p1_chunk(xb, yb, fb, sv, mstat, lstat, b, grp)

                @pl.when(c + 2 < _NCH)
                def _():
                    x_copy(c + 2, b).start()
                    f_copy(c + 2, b).start()
            return carry

        lax.fori_loop(0, _NCH // 2, pair, 0)
        c_last = _NCH - 1
        x_copy(c_last, 0).wait()
        f_copy(c_last, 0).wait()
        _p1_chunk(xb, yb, fb, sv, mstat, lstat, 0, grp)

        pltpu.sync_copy(xt_hbm.at[pl.ds(rowg * _VT, _G * _VT)], xtb)
        pltpu.sync_copy(ft_hbm, ftb)
        ftv = [ftb[pl.ds(0, 16)], ftb[pl.ds(16, 16)]]
        for r in range(_G):
            lr = grp * _G + r
            macc = mstat[pl.ds(lr * 16, 16)]
            y0 = xtb[pl.ds(r * _VT, 16)] + sv * ftv[0]
            y1 = xtb[pl.ds(r * _VT + 16, 16)] + sv * ftv[1]
            mnew = jnp.maximum(macc, jnp.maximum(y0, y1))
            lsum = jnp.exp(y0 - mnew) + jnp.exp(y1 - mnew)
            lold = lstat[pl.ds(lr * 16, 16)]
            lstat[pl.ds(lr * 16, 16)] = lold * jnp.exp(macc - mnew) + lsum
            mstat[pl.ds(lr * 16, 16)] = mnew

    pltpu.sync_copy(mstat, m_hbm.at[pl.ds(row0 * 16, _RPT * 16)])
    pltpu.sync_copy(lstat, l_hbm.at[pl.ds(row0 * 16, _RPT * 16)])


def _p1_sc(x, xtail, f, ftail, s16):
    fn = pl.kernel(
        _p1_body,
        out_type=(
            jax.ShapeDtypeStruct((_B * 16,), jnp.float32),
            jax.ShapeDtypeStruct((_B * 16,), jnp.float32),
        ),
        mesh=_mesh(),
        scratch_types=[
            pltpu.VMEM((2, _G, _C), jnp.float32),
            pltpu.VMEM((2, _G, _C), jnp.float32),
            pltpu.VMEM((2, _C), jnp.float32),
            pltpu.VMEM((16,), jnp.float32),
            pltpu.VMEM((_G * _VT,), jnp.float32),
            pltpu.VMEM((_VT,), jnp.float32),
            pltpu.VMEM((_RPT * 16,), jnp.float32),
            pltpu.VMEM((_RPT * 16,), jnp.float32),
            pltpu.SemaphoreType.DMA((2,)),
            pltpu.SemaphoreType.DMA((2,)),
        ],
        compiler_params=_SC_PARAMS,
    )
    return fn(x, xtail, f, ftail, s16)


def _p2_body(x_hbm, xt_hbm, f_hbm, ft_hbm, s_hbm, m_hbm, l_hbm,
             o_hbm, ot_hbm,
             xb, ob, fb, sb, xtb, ftb, mlstat, statb, xsem, fsem, osem):
    wid = lax.axis_index("s") * 2 + lax.axis_index("c")
    row0 = wid * _RPT
    pltpu.sync_copy(s_hbm, sb)
    sv = sb[...]

    ln2 = jnp.float32(0.69314718)
    for half in range(2):
        pltpu.sync_copy(
            m_hbm.at[pl.ds((row0 + half * _G) * 16, _G * 16)], statb)
        for r in range(_G):
            lr = half * _G + r
            m16 = statb[pl.ds(r * 16, 16)]
            mlstat[pl.ds(lr * 16, 16)] = m16
        pltpu.sync_copy(
            l_hbm.at[pl.ds((row0 + half * _G) * 16, _G * 16)], statb)
        for r in range(_G):
            lr = half * _G + r
            m16 = mlstat[pl.ds(lr * 16, 16)]
            l16 = statb[pl.ds(r * 16, 16)]
            mstar = jnp.max(m16)
            lsum = jnp.sum(l16 * jnp.exp(m16 - mstar))
            lv = jnp.full((16,), lsum, jnp.float32)
            bits = lax.bitcast_convert_type(lv, jnp.int32)
            e = ((bits >> 23) & 255) - 127
            mant = lax.bitcast_convert_type(
                (bits & 0x007FFFFF) | 0x3F800000, jnp.float32)
            u = mant - 1.0
            t = e.astype(jnp.float32) * ln2 + u * (
                1.0 - 0.5 * u + 0.3333334 * (u * u))
            for _ in range(3):
                t = t + lv * jnp.exp(-t) - 1.0
            mlstat[pl.ds(lr * 16, 16)] = mstar + t

    for grp in range(_NG):
        rowg = row0 + grp * _G

        def x_copy(c, b):
            return pltpu.make_async_copy(
                x_hbm.at[pl.ds(rowg, _G), pl.ds(c * _C, _C)],
                xb.at[b], xsem.at[b])

        def f_copy(c, b):
            return pltpu.make_async_copy(
                f_hbm.at[pl.ds(c * _C, _C)], fb.at[b], fsem.at[b])

        def o_copy(c, b):
            return pltpu.make_async_copy(
                ob.at[b],
                o_hbm.at[pl.ds(rowg, _G), pl.ds(c * _C, _C)], osem.at[b])

        def chunk(b, grp_):
            for r in range(_G):
                mlv = mlstat[pl.ds((grp_ * _G + r) * 16, 16)]

                def sw(v, carry):
                    xv = xb[b, r, pl.ds(v * 16, 16)]
                    fv = fb[b, pl.ds(v * 16, 16)]
                    ob[b, r, pl.ds(v * 16, 16)] = xv + sv * fv - mlv
                    return carry

                lax.fori_loop(0, _NV, sw, 0)

        x_copy(0, 0).start()
        f_copy(0, 0).start()
        x_copy(1, 1).start()
        f_copy(1, 1).start()

        def pair(t, carry):
            for b in range(2):
                c = 2 * t + b
                x_copy(c, b).wait()
                f_copy(c, b).wait()

                @pl.when(c >= 2)
                def _():
                    o_copy(c - 2, b).wait()

                chunk(b, grp)
                o_copy(c, b).start()

                @pl.when(c + 2 < _NCH)
                def _():
                    x_copy(c + 2, b).start()
                    f_copy(c + 2, b).start()
            return carry

        lax.fori_loop(0, _NCH // 2, pair, 0)
        c_last = _NCH - 1
        x_copy(c_last, 0).wait()
        f_copy(c_last, 0).wait()
        o_copy(c_last - 2, 0).wait()
        chunk(0, grp)
        o_copy(c_last, 0).start()
        o_copy(c_last - 1, 1).wait()
        o_copy(c_last, 0).wait()

        pltpu.sync_copy(xt_hbm.at[pl.ds(rowg * _VT, _G * _VT)], xtb)
        pltpu.sync_copy(ft_hbm, ftb)
        ftv = [ftb[pl.ds(0, 16)], ftb[pl.ds(16, 16)]]
        for r in range(_G):
            mlv = mlstat[pl.ds((grp * _G + r) * 16, 16)]
            xtb[pl.ds(r * _VT, 16)] = (
                xtb[pl.ds(r * _VT, 16)] + sv * ftv[0] - mlv)
            xtb[pl.ds(r * _VT + 16, 16)] = (
                xtb[pl.ds(r * _VT + 16, 16)] + sv * ftv[1] - mlv)
        pltpu.sync_copy(xtb, ot_hbm.at[pl.ds(rowg * _VT, _G * _VT)])


def _p2_sc(x, xtail, f, ftail, s16, m_all, l_all):
    fn = pl.kernel(
        _p2_body,
        out_type=(
            jax.ShapeDtypeStruct((_B, _V), jnp.float32),
            jax.ShapeDtypeStruct((_B * _VT,), jnp.float32),
        ),
        mesh=_mesh(),
        scratch_types=[
            pltpu.VMEM((2, _G, _C), jnp.float32),
            pltpu.VMEM((2, _G, _C), jnp.float32),
            pltpu.VMEM((2, _C), jnp.float32),
            pltpu.VMEM((16,), jnp.float32),
            pltpu.VMEM((_G * _VT,), jnp.float32),
            pltpu.VMEM((_VT,), jnp.float32),
            pltpu.VMEM((_RPT * 16,), jnp.float32),
            pltpu.VMEM((_G * 16,), jnp.float32),
            pltpu.SemaphoreType.DMA((2,)),
            pltpu.SemaphoreType.DMA((2,)),
            pltpu.SemaphoreType.DMA((2,)),
        ],
        compiler_params=_SC_PARAMS,
    )
    return fn(x, xtail, f, ftail, s16, m_all, l_all)


def kernel(decoder_output, decoded_words, pattern, w1, lv_table, i, j, batch_size):
    n = jnp.asarray(i, dtype=jnp.int32) * 7 + jnp.asarray(j, dtype=jnp.int32)
    cond = (n > 0) & (jnp.asarray(j) < 7) & (jnp.asarray(i) < 4)

    nd = decoded_words.shape[1]
    idx = jnp.pad(decoded_words[0], (0, 32 - nd))
    prow = jnp.pad(jnp.take(pattern, n, axis=0), (0, 4))
    w1b = jnp.broadcast_to(jnp.where(cond, w1[0], 0.0), (16,)).astype(jnp.float32)
    table = lv_table[:, 0]

    xtail = decoder_output[:, _VA:].reshape(-1)
    ftail = table[_VA:]

    s16 = _s_sc(idx, prow, w1b, table)
    m_all, l_all = _p1_sc(decoder_output, xtail, table, ftail, s16)
    o_main, otail = _p2_sc(decoder_output, xtail, table, ftail, s16,
                           m_all, l_all)
    return lax.dynamic_update_slice(o_main, otail.reshape(_B, _VT), (0, _VA))


# speedup vs baseline: 4.1012x; 4.1012x over previous
import functools

import jax
import jax.numpy as jnp
from jax import lax
from jax.experimental import pallas as pl
from jax.experimental.pallas import tpu as pltpu
from jax.experimental.pallas import tpu_sc as plsc

_V = 100000
_BB = 32


def _s_body(idx_hbm, prow_hbm, w1_hbm, table_hbm, s_hbm,
            idx_v, prow_v, w1_v, table_v, out_v):
    @pl.when((lax.axis_index("c") == 0) & (lax.axis_index("s") == 0))
    def _():
        pltpu.sync_copy(idx_hbm, idx_v)
        pltpu.sync_copy(prow_hbm, prow_v)
        pltpu.sync_copy(w1_hbm, w1_v)
        pltpu.sync_copy(table_hbm, table_v)
        acc = jnp.zeros((16,), jnp.float32)
        for h in range(2):
            iv = idx_v[pl.ds(h * 16, 16)]
            vals = plsc.load_gather(table_v, [iv])
            acc = acc + vals * prow_v[pl.ds(h * 16, 16)]
        tot = jnp.sum(acc)
        out_v[...] = tot * w1_v[...]
        pltpu.sync_copy(out_v, s_hbm)


def _s_sc(idx, prow, w1b, table):
    mesh = plsc.VectorSubcoreMesh(core_axis_name="c", subcore_axis_name="s")
    fn = pl.kernel(
        _s_body,
        out_type=jax.ShapeDtypeStruct((16,), jnp.float32),
        mesh=mesh,
        scratch_types=[
            pltpu.VMEM((32,), jnp.int32),
            pltpu.VMEM((32,), jnp.float32),
            pltpu.VMEM((16,), jnp.float32),
            pltpu.VMEM((_V,), jnp.float32),
            pltpu.VMEM((16,), jnp.float32),
        ],
        compiler_params=pltpu.CompilerParams(needs_layout_passes=False),
    )
    return fn(idx, prow, w1b, table)


def _main_body(s_ref, x_ref, f_ref, o_ref):
    s = s_ref[0]
    y = x_ref[...] + s * f_ref[...]
    m = jnp.max(y, axis=1, keepdims=True)
    l = jnp.log(jnp.sum(jnp.exp(y - m), axis=1, keepdims=True))
    o_ref[...] = y - m - l


def _main(s, x, f):
    batch, vocab = x.shape
    return pl.pallas_call(
        _main_body,
        grid=(batch // _BB,),
        in_specs=[
            pl.BlockSpec(memory_space=pltpu.SMEM),
            pl.BlockSpec((_BB, vocab), lambda b: (b, 0)),
            pl.BlockSpec((1, vocab), lambda b: (0, 0)),
        ],
        out_specs=pl.BlockSpec((_BB, vocab), lambda b: (b, 0)),
        out_shape=jax.ShapeDtypeStruct((batch, vocab), jnp.float32),
        compiler_params=pltpu.CompilerParams(
            dimension_semantics=("arbitrary",),
            vmem_limit_bytes=100 * 1024 * 1024,
        ),
    )(s, x, f)


def kernel(decoder_output, decoded_words, pattern, w1, lv_table, i, j, batch_size):
    n = jnp.asarray(i, dtype=jnp.int32) * 7 + jnp.asarray(j, dtype=jnp.int32)
    cond = (n > 0) & (jnp.asarray(j) < 7) & (jnp.asarray(i) < 4)

    nd = decoded_words.shape[1]
    idx = jnp.pad(decoded_words[0], (0, 32 - nd))
    prow = jnp.pad(jnp.take(pattern, n, axis=0), (0, 4))
    w1b = jnp.broadcast_to(jnp.where(cond, w1[0], 0.0), (16,)).astype(jnp.float32)
    table = lv_table[:, 0]

    s16 = _s_sc(idx, prow, w1b, table)
    s = s16[:1]

    f = lv_table.reshape(1, -1)
    return _main(s, decoder_output, f)
